# baseline (speedup 1.0000x reference)
import jax
import jax.numpy as jnp
from jax.experimental import pallas as pl
from jax.experimental.pallas import tpu as pltpu

_N = 8732
_NP = 8832
_C = 21
_NC = _C - 1
_TOP = 200
_CONF_TH = 0.01
_NMS_TH = 0.45
_BIG = 2 ** 30


def _ssd_body(conf_ref, loc_ref, pri_ref, out_ref,
              scores_ref, bt_ref, sels_ref,
              sx1_ref, sy1_ref, sx2_ref, sy2_ref, keep_ref, area_ref):
    conf = conf_ref[0]
    cmax = jnp.max(conf, axis=0, keepdims=True)
    e = jnp.exp(conf - cmax)
    den = jnp.sum(e, axis=0, keepdims=True)
    sc = e[1:, :] / den
    valid = jax.lax.broadcasted_iota(jnp.int32, (_NC, _NP), 1) < _N
    scores_ref[...] = jnp.where(valid, jnp.where(sc > _CONF_TH, sc, 0.0), -1.0)

    l = loc_ref[0]
    p = pri_ref[...]
    pxy, pwh = p[:, 0:2], p[:, 2:4]
    cxy = pxy + l[:, 0:2] * 0.1 * pwh
    wh = pwh * jnp.exp(l[:, 2:4] * 0.2)
    bt_ref[:, 0:2] = cxy - wh / 2.0
    bt_ref[:, 2:4] = cxy + wh / 2.0

    sels_ref[...] = jnp.zeros((_NC, _TOP), jnp.float32)
    sx1_ref[...] = jnp.zeros((_NC, _TOP), jnp.float32)
    sy1_ref[...] = jnp.zeros((_NC, _TOP), jnp.float32)
    sx2_ref[...] = jnp.zeros((_NC, _TOP), jnp.float32)
    sy2_ref[...] = jnp.zeros((_NC, _TOP), jnp.float32)
    keep_ref[...] = jnp.ones((_NC, _TOP), jnp.float32)

    def sel_body(k, _):
        coln = jax.lax.broadcasted_iota(jnp.int32, (_NC, _NP), 1)
        s = scores_ref[...]
        m = jnp.max(s, axis=1, keepdims=True)
        cand = jnp.where(s == m, coln, _BIG)
        idx = jnp.min(cand, axis=1, keepdims=True)
        chosen = coln == idx
        oh = jax.lax.broadcasted_iota(jnp.int32, (_NC, _TOP), 1) == k
        sels_ref[...] += jnp.where(oh, m, 0.0)
        g = jnp.dot(chosen.astype(jnp.float32), bt_ref[...],
                    precision=jax.lax.Precision.HIGHEST,
                    preferred_element_type=jnp.float32)
        sx1_ref[...] += jnp.where(oh, g[:, 0:1], 0.0)
        sy1_ref[...] += jnp.where(oh, g[:, 1:2], 0.0)
        sx2_ref[...] += jnp.where(oh, g[:, 2:3], 0.0)
        sy2_ref[...] += jnp.where(oh, g[:, 3:4], 0.0)
        scores_ref[...] = jnp.where(chosen, -1.0, s)
        return 0

    jax.lax.fori_loop(0, _TOP, sel_body, 0)

    x1 = sx1_ref[...]
    y1 = sy1_ref[...]
    x2 = sx2_ref[...]
    y2 = sy2_ref[...]
    area_ref[...] = (jnp.maximum(x2 - x1, 0.0) * jnp.maximum(y2 - y1, 0.0))

    def nms_body(i, _):
        col = jax.lax.broadcasted_iota(jnp.int32, (_NC, _TOP), 1)
        oh = col == i
        keep = keep_ref[...]
        bx1 = sx1_ref[...]
        by1 = sy1_ref[...]
        bx2 = sx2_ref[...]
        by2 = sy2_ref[...]
        area = area_ref[...]

        def ext(a):
            return jnp.sum(jnp.where(oh, a, 0.0), axis=1, keepdims=True)

        xi1, yi1 = ext(bx1), ext(by1)
        xi2, yi2 = ext(bx2), ext(by2)
        ai, ki = ext(area), ext(keep)
        inter = (jnp.maximum(jnp.minimum(xi2, bx2) - jnp.maximum(xi1, bx1), 0.0)
                 * jnp.maximum(jnp.minimum(yi2, by2) - jnp.maximum(yi1, by1), 0.0))
        union = ai + area - inter
        sup = (inter > _NMS_TH * (union + 1e-9)) & (col > i) & (ki > 0.5)
        keep_ref[...] = jnp.where(sup, 0.0, keep)
        return 0

    jax.lax.fori_loop(0, _TOP, nms_body, 0)

    out_ref[0, :, 0, :] = sels_ref[...] * keep_ref[...]
    out_ref[0, :, 1, :] = sx1_ref[...]
    out_ref[0, :, 2, :] = sy1_ref[...]
    out_ref[0, :, 3, :] = sx2_ref[...]
    out_ref[0, :, 4, :] = sy2_ref[...]


@jax.jit
def kernel(locations, confidences, priors):
    b = locations.shape[0]
    pad = _NP - _N
    conf_t = jnp.pad(jnp.transpose(confidences, (0, 2, 1)),
                     ((0, 0), (0, 0), (0, pad)))
    loc_p = jnp.pad(locations, ((0, 0), (0, pad), (0, 0)))
    pri_p = jnp.pad(priors, ((0, pad), (0, 0)))

    out = pl.pallas_call(
        _ssd_body,
        grid=(b,),
        in_specs=[
            pl.BlockSpec((1, _C, _NP), lambda i: (i, 0, 0)),
            pl.BlockSpec((1, _NP, 4), lambda i: (i, 0, 0)),
            pl.BlockSpec((_NP, 4), lambda i: (0, 0)),
        ],
        out_specs=pl.BlockSpec((1, _NC, 5, _TOP), lambda i: (i, 0, 0, 0)),
        out_shape=jax.ShapeDtypeStruct((b, _NC, 5, _TOP), jnp.float32),
        compiler_params=pltpu.CompilerParams(
            dimension_semantics=("parallel",)),
        scratch_shapes=[
            pltpu.VMEM((_NC, _NP), jnp.float32),
            pltpu.VMEM((_NP, 4), jnp.float32),
            pltpu.VMEM((_NC, _TOP), jnp.float32),
            pltpu.VMEM((_NC, _TOP), jnp.float32),
            pltpu.VMEM((_NC, _TOP), jnp.float32),
            pltpu.VMEM((_NC, _TOP), jnp.float32),
            pltpu.VMEM((_NC, _TOP), jnp.float32),
            pltpu.VMEM((_NC, _TOP), jnp.float32),
            pltpu.VMEM((_NC, _TOP), jnp.float32),
        ],
    )(conf_t, loc_p, pri_p)

    return jnp.transpose(out, (0, 1, 3, 2))


# speedup vs baseline: 2.2380x; 2.2380x over previous
import jax
import jax.numpy as jnp
from jax.experimental import pallas as pl
from jax.experimental.pallas import tpu as pltpu

_N = 8732
_NP = 8832
_C = 21
_NC = _C - 1
_TOP = 200
_CONF_TH = 0.01
_NMS_TH = 0.45
_BIG = 2 ** 30


def _ssd_body(conf_ref, loc_ref, pri_ref, out_ref,
              scores_ref, bt_ref, sels_ref, sidx_ref,
              sx1_ref, sy1_ref, sx2_ref, sy2_ref, keep_ref, area_ref):
    conf = conf_ref[0]
    cmax = jnp.max(conf, axis=0, keepdims=True)
    e = jnp.exp(conf - cmax)
    den = jnp.sum(e, axis=0, keepdims=True)
    sc = e[1:, :] / den
    valid = jax.lax.broadcasted_iota(jnp.int32, (_NC, _NP), 1) < _N
    scores_ref[...] = jnp.where(valid, jnp.where(sc > _CONF_TH, sc, 0.0), -1.0)

    l = loc_ref[0]
    p = pri_ref[...]
    pxy, pwh = p[:, 0:2], p[:, 2:4]
    cxy = pxy + l[:, 0:2] * 0.1 * pwh
    wh = pwh * jnp.exp(l[:, 2:4] * 0.2)
    bt_ref[:, 0:2] = cxy - wh / 2.0
    bt_ref[:, 2:4] = cxy + wh / 2.0

    sels_ref[...] = jnp.zeros((_NC, _TOP), jnp.float32)
    sidx_ref[...] = jnp.zeros((_NC, _TOP), jnp.int32)
    keep_ref[...] = jnp.ones((_NC, _TOP), jnp.float32)

    def sel_body(k, _):
        coln = jax.lax.broadcasted_iota(jnp.int32, (_NC, _NP), 1)
        s = scores_ref[...]
        m = jnp.max(s, axis=1, keepdims=True)
        cand = jnp.where(s == m, coln, _BIG)
        idx = jnp.min(cand, axis=1, keepdims=True)
        chosen = coln == idx
        oh = jax.lax.broadcasted_iota(jnp.int32, (_NC, _TOP), 1) == k
        sels_ref[...] += jnp.where(oh, m, 0.0)
        sidx_ref[...] += jnp.where(oh, idx, 0)
        scores_ref[...] = jnp.where(chosen, -1.0, s)
        return 0

    jax.lax.fori_loop(0, _TOP, sel_body, 0)

    def gather_body(c, _):
        row = sidx_ref[pl.ds(c, 1), :]
        rt = jnp.transpose(row, (1, 0))
        pm = jax.lax.broadcasted_iota(jnp.int32, (_TOP, _NP), 1) == rt
        pf = pm.astype(jnp.float32)
        g = jnp.dot(pf, bt_ref[...],
                    precision=jax.lax.Precision.HIGHEST,
                    preferred_element_type=jnp.float32)
        gt = jnp.transpose(g, (1, 0))
        sx1_ref[pl.ds(c, 1), :] = gt[0:1, :]
        sy1_ref[pl.ds(c, 1), :] = gt[1:2, :]
        sx2_ref[pl.ds(c, 1), :] = gt[2:3, :]
        sy2_ref[pl.ds(c, 1), :] = gt[3:4, :]
        return 0

    jax.lax.fori_loop(0, _NC, gather_body, 0)

    x1 = sx1_ref[...]
    y1 = sy1_ref[...]
    x2 = sx2_ref[...]
    y2 = sy2_ref[...]
    area_ref[...] = (jnp.maximum(x2 - x1, 0.0) * jnp.maximum(y2 - y1, 0.0))

    def nms_body(i, _):
        col = jax.lax.broadcasted_iota(jnp.int32, (_NC, _TOP), 1)
        oh = col == i
        keep = keep_ref[...]
        bx1 = sx1_ref[...]
        by1 = sy1_ref[...]
        bx2 = sx2_ref[...]
        by2 = sy2_ref[...]
        area = area_ref[...]

        def ext(a):
            return jnp.sum(jnp.where(oh, a, 0.0), axis=1, keepdims=True)

        xi1, yi1 = ext(bx1), ext(by1)
        xi2, yi2 = ext(bx2), ext(by2)
        ai, ki = ext(area), ext(keep)
        inter = (jnp.maximum(jnp.minimum(xi2, bx2) - jnp.maximum(xi1, bx1), 0.0)
                 * jnp.maximum(jnp.minimum(yi2, by2) - jnp.maximum(yi1, by1), 0.0))
        union = ai + area - inter
        sup = (inter > _NMS_TH * (union + 1e-9)) & (col > i) & (ki > 0.5)
        keep_ref[...] = jnp.where(sup, 0.0, keep)
        return 0

    jax.lax.fori_loop(0, _TOP, nms_body, 0)

    out_ref[0, :, 0, :] = sels_ref[...] * keep_ref[...]
    out_ref[0, :, 1, :] = sx1_ref[...]
    out_ref[0, :, 2, :] = sy1_ref[...]
    out_ref[0, :, 3, :] = sx2_ref[...]
    out_ref[0, :, 4, :] = sy2_ref[...]


@jax.jit
def kernel(locations, confidences, priors):
    b = locations.shape[0]
    pad = _NP - _N
    conf_t = jnp.pad(jnp.transpose(confidences, (0, 2, 1)),
                     ((0, 0), (0, 0), (0, pad)))
    loc_p = jnp.pad(locations, ((0, 0), (0, pad), (0, 0)))
    pri_p = jnp.pad(priors, ((0, pad), (0, 0)))

    out = pl.pallas_call(
        _ssd_body,
        grid=(b,),
        in_specs=[
            pl.BlockSpec((1, _C, _NP), lambda i: (i, 0, 0)),
            pl.BlockSpec((1, _NP, 4), lambda i: (i, 0, 0)),
            pl.BlockSpec((_NP, 4), lambda i: (0, 0)),
        ],
        out_specs=pl.BlockSpec((1, _NC, 5, _TOP), lambda i: (i, 0, 0, 0)),
        out_shape=jax.ShapeDtypeStruct((b, _NC, 5, _TOP), jnp.float32),
        compiler_params=pltpu.CompilerParams(
            dimension_semantics=("parallel",)),
        scratch_shapes=[
            pltpu.VMEM((_NC, _NP), jnp.float32),
            pltpu.VMEM((_NP, 4), jnp.float32),
            pltpu.VMEM((_NC, _TOP), jnp.float32),
            pltpu.VMEM((_NC, _TOP), jnp.int32),
            pltpu.VMEM((_NC, _TOP), jnp.float32),
            pltpu.VMEM((_NC, _TOP), jnp.float32),
            pltpu.VMEM((_NC, _TOP), jnp.float32),
            pltpu.VMEM((_NC, _TOP), jnp.float32),
            pltpu.VMEM((_NC, _TOP), jnp.float32),
            pltpu.VMEM((_NC, _TOP), jnp.float32),
        ],
    )(conf_t, loc_p, pri_p)

    return jnp.transpose(out, (0, 1, 3, 2))


# speedup vs baseline: 2.6732x; 1.1945x over previous
import jax
import jax.numpy as jnp
from jax.experimental import pallas as pl
from jax.experimental.pallas import tpu as pltpu

_N = 8732
_C = 21
_NC = _C - 1
_TOP = 200
_CONF_TH = 0.01
_NMS_TH = 0.45
_BIG = 2 ** 30


def _ssd_body(conf_ref, loc_ref, pri_ref, out_ref,
              scores_ref, boxes_ref, sels_ref, sx1_ref, sy1_ref,
              sx2_ref, sy2_ref, keep_ref, area_ref):
    conf = conf_ref[0]
    cmax = jnp.max(conf, axis=0, keepdims=True)
    e = jnp.exp(conf - cmax)
    den = jnp.sum(e, axis=0, keepdims=True)
    sc = e[1:, :] / den
    scores_ref[...] = jnp.where(sc > _CONF_TH, sc, 0.0)

    loc = loc_ref[0]
    pri = pri_ref[...]
    px, py = pri[0:1, :], pri[1:2, :]
    pw, ph = pri[2:3, :], pri[3:4, :]
    cx = px + loc[0:1, :] * 0.1 * pw
    cy = py + loc[1:2, :] * 0.1 * ph
    w = pw * jnp.exp(loc[2:3, :] * 0.2)
    h = ph * jnp.exp(loc[3:4, :] * 0.2)
    boxes_ref[0:1, :] = cx - w / 2.0
    boxes_ref[1:2, :] = cy - h / 2.0
    boxes_ref[2:3, :] = cx + w / 2.0
    boxes_ref[3:4, :] = cy + h / 2.0

    zero_t = jnp.zeros((_NC, _TOP), jnp.float32)
    sels_ref[...] = zero_t
    sx1_ref[...] = zero_t
    sy1_ref[...] = zero_t
    sx2_ref[...] = zero_t
    sy2_ref[...] = zero_t
    keep_ref[...] = jnp.ones((_NC, _TOP), jnp.float32)

    def sel_body(k, _):
        coln = jax.lax.broadcasted_iota(jnp.int32, (_NC, _N), 1)
        s = scores_ref[...]
        m = jnp.max(s, axis=1, keepdims=True)
        cand = jnp.where(s == m, coln, _BIG)
        idx = jnp.min(cand, axis=1, keepdims=True)
        chosen = coln == idx
        oh = jax.lax.broadcasted_iota(jnp.int32, (_NC, _TOP), 1) == k
        sels_ref[...] += jnp.where(oh, m, 0.0)
        boxes = boxes_ref[...]
        for j, ref in enumerate((sx1_ref, sy1_ref, sx2_ref, sy2_ref)):
            v = jnp.sum(jnp.where(chosen, boxes[j:j + 1, :], 0.0),
                        axis=1, keepdims=True)
            ref[...] += jnp.where(oh, v, 0.0)
        scores_ref[...] = jnp.where(chosen, -1.0, s)
        return 0

    jax.lax.fori_loop(0, _TOP, sel_body, 0)

    x1 = sx1_ref[...]
    y1 = sy1_ref[...]
    x2 = sx2_ref[...]
    y2 = sy2_ref[...]
    area_ref[...] = (jnp.maximum(x2 - x1, 0.0) * jnp.maximum(y2 - y1, 0.0))

    def nms_body(i, _):
        col = jax.lax.broadcasted_iota(jnp.int32, (_NC, _TOP), 1)
        oh = col == i
        keep = keep_ref[...]
        bx1 = sx1_ref[...]
        by1 = sy1_ref[...]
        bx2 = sx2_ref[...]
        by2 = sy2_ref[...]
        area = area_ref[...]

        def ext(a):
            return jnp.sum(jnp.where(oh, a, 0.0), axis=1, keepdims=True)

        xi1, yi1 = ext(bx1), ext(by1)
        xi2, yi2 = ext(bx2), ext(by2)
        ai, ki = ext(area), ext(keep)
        inter = (jnp.maximum(jnp.minimum(xi2, bx2) - jnp.maximum(xi1, bx1), 0.0)
                 * jnp.maximum(jnp.minimum(yi2, by2) - jnp.maximum(yi1, by1), 0.0))
        union = ai + area - inter
        sup = (inter > _NMS_TH * (union + 1e-9)) & (col > i) & (ki > 0.5)
        keep_ref[...] = jnp.where(sup, 0.0, keep)
        return 0

    jax.lax.fori_loop(0, _TOP, nms_body, 0)

    out_ref[0, :, 0, :] = sels_ref[...] * keep_ref[...]
    out_ref[0, :, 1, :] = sx1_ref[...]
    out_ref[0, :, 2, :] = sy1_ref[...]
    out_ref[0, :, 3, :] = sx2_ref[...]
    out_ref[0, :, 4, :] = sy2_ref[...]


@jax.jit
def kernel(locations, confidences, priors):
    b = locations.shape[0]
    conf_t = jnp.transpose(confidences, (0, 2, 1))
    loc_t = jnp.transpose(locations, (0, 2, 1))
    pri_t = priors.T

    out = pl.pallas_call(
        _ssd_body,
        grid=(b,),
        in_specs=[
            pl.BlockSpec((1, _C, _N), lambda i: (i, 0, 0)),
            pl.BlockSpec((1, 4, _N), lambda i: (i, 0, 0)),
            pl.BlockSpec((4, _N), lambda i: (0, 0)),
        ],
        out_specs=pl.BlockSpec((1, _NC, 5, _TOP), lambda i: (i, 0, 0, 0)),
        out_shape=jax.ShapeDtypeStruct((b, _NC, 5, _TOP), jnp.float32),
        compiler_params=pltpu.CompilerParams(
            dimension_semantics=("parallel",)),
        scratch_shapes=[
            pltpu.VMEM((_NC, _N), jnp.float32),
            pltpu.VMEM((4, _N), jnp.float32),
            pltpu.VMEM((_NC, _TOP), jnp.float32),
            pltpu.VMEM((_NC, _TOP), jnp.float32),
            pltpu.VMEM((_NC, _TOP), jnp.float32),
            pltpu.VMEM((_NC, _TOP), jnp.float32),
            pltpu.VMEM((_NC, _TOP), jnp.float32),
            pltpu.VMEM((_NC, _TOP), jnp.float32),
            pltpu.VMEM((_NC, _TOP), jnp.float32),
        ],
    )(conf_t, loc_t, pri_t)

    return jnp.transpose(out, (0, 1, 3, 2))
